# baseline (speedup 1.0000x reference)
import functools

import jax
import jax.numpy as jnp
from jax import lax
from jax.experimental import pallas as pl
from jax.experimental.pallas import tpu as pltpu
from jax.experimental.pallas import tpu_sc as plsc

NUM_CORES = 2
NUM_SUBCORES = 16
LANES = 16
NW = NUM_CORES * NUM_SUBCORES


def _sc_body(R, D, b_per_w, emb_hbm, idx_hbm, table_hbm, out_hbm,
             idx_v, ebuf, rows, sem):
    c = lax.axis_index("c")
    s = lax.axis_index("s")
    wid = s * NUM_CORES + c
    base = wid * b_per_w
    n_chunks = b_per_w // R

    pltpu.sync_copy(idx_hbm.at[wid], idx_v)

    for j in range(n_chunks):
        rb = base + j * R
        gather = pltpu.async_copy(table_hbm.at[idx_v.at[j]], rows, sem)
        pltpu.sync_copy(emb_hbm.at[pl.ds(rb, R)], ebuf)
        gather.wait()

        def add_row(r, _):
            for cc in range(D // LANES):
                sl = pl.ds(cc * LANES, LANES)
                ebuf[r, sl] = ebuf[r, sl] + rows[r, sl]
            return _

        lax.fori_loop(0, R, add_row, None)
        pltpu.sync_copy(ebuf, out_hbm.at[pl.ds(rb, R)])


def kernel(batch_Phrase_emb, Phrase_type_ids, phrase_attribute_emb_all):
    B, D = batch_Phrase_emb.shape
    b_per_w = B // NW
    R = 64
    n_chunks = b_per_w // R

    idx = Phrase_type_ids.astype(jnp.int32).reshape(NW, n_chunks, R)

    mesh = plsc.VectorSubcoreMesh(
        core_axis_name="c", subcore_axis_name="s",
        num_cores=NUM_CORES, num_subcores=NUM_SUBCORES)
    f = pl.kernel(
        functools.partial(_sc_body, R, D, b_per_w),
        out_type=jax.ShapeDtypeStruct((B, D), jnp.float32),
        mesh=mesh,
        scratch_types=[
            pltpu.VMEM((n_chunks, R), jnp.int32),
            pltpu.VMEM((R, D), jnp.float32),
            pltpu.VMEM((R, D), jnp.float32),
            pltpu.SemaphoreType.DMA,
        ],
    )
    return f(batch_Phrase_emb, idx, phrase_attribute_emb_all)
